# baseline (speedup 1.0000x reference)
import functools

import jax
import jax.numpy as jnp
from jax import lax
from jax.experimental import pallas as pl
from jax.experimental.pallas import tpu as pltpu
from jax.experimental.pallas import tpu_sc as plsc

B = 128
V = 100000
NC = 2
NS = 16
NW = NC * NS
ROWS_PER_W = B // NW
LANES = 16
CHUNK = 20000
NCHUNK = V // CHUNK
ITERS = CHUNK // LANES

_LOG_NOISE = None


def _log_noise_flat():
    global _LOG_NOISE
    if _LOG_NOISE is None:
        noise = jax.random.exponential(jax.random.key(42), (B, V), dtype=jnp.float32)
        noise = jnp.maximum(noise, 1e-10)
        _LOG_NOISE = jnp.log(noise).reshape(-1)
    return _LOG_NOISE


def _sc_body(logits_hbm, ln_hbm, aux_hbm, out_hbm,
             la0, la1, ln0, ln1, aux_v, res_v,
             sla0, sla1, sln0, sln1):
    wid = lax.axis_index("s") * NC + lax.axis_index("c")
    pltpu.sync_copy(aux_hbm.at[wid], aux_v)
    avec = aux_v[...]
    lane = lax.broadcasted_iota(jnp.int32, (LANES,), 0)
    neg_inf = jnp.float32(-jnp.inf)

    gather_dn = lax.GatherDimensionNumbers(
        offset_dims=(), collapsed_slice_dims=(0,), start_index_map=(0,))

    def perm(x, idx):
        return lax.gather(x, idx[:, None], gather_dn, (1,),
                          mode=lax.GatherScatterMode.PROMISE_IN_BOUNDS)

    def splat(x, j):
        return perm(x, jnp.full((LANES,), j, jnp.int32))

    def allmax(x):
        for sh in (1, 2, 4, 8):
            x = jnp.maximum(x, perm(x, lane ^ sh))
        return x

    def allmin(x):
        for sh in (1, 2, 4, 8):
            x = jnp.minimum(x, perm(x, lane ^ sh))
        return x

    nmask = splat(avec, ROWS_PER_W)

    la_bufs = (la0, la1)
    ln_bufs = (ln0, ln1)
    la_sems = (sla0, sla1)
    ln_sems = (sln0, sln1)

    res = jnp.zeros((LANES,), jnp.int32)
    for r in range(ROWS_PER_W):
        inv_t = splat(avec, r)
        row_base = (wid * ROWS_PER_W + r) * V

        copies = {}

        def start(c):
            b = c % 2
            copies[c] = (
                pltpu.async_copy(
                    logits_hbm.at[pl.ds(row_base + c * CHUNK, CHUNK)],
                    la_bufs[b], la_sems[b]),
                pltpu.async_copy(
                    ln_hbm.at[pl.ds(row_base + c * CHUNK, CHUNK)],
                    ln_bufs[b], ln_sems[b]),
            )

        start(0)
        best_val = jnp.full((LANES,), neg_inf, jnp.float32)
        best_idx = jnp.zeros((LANES,), jnp.int32)
        for c in range(NCHUNK):
            if c + 1 < NCHUNK:
                start(c + 1)
            for cp in copies.pop(c):
                cp.wait()
            la = la_bufs[c % 2]
            ln = ln_bufs[c % 2]

            def body(i, carry, la=la, ln=ln):
                bv, bi, idx = carry
                a = la[pl.ds(i * LANES, LANES)]
                n = ln[pl.ds(i * LANES, LANES)]
                s = a * inv_t - nmask * n
                take = s > bv
                bv = jnp.where(take, s, bv)
                bi = jnp.where(take, idx, bi)
                return bv, bi, idx + LANES

            best_val, best_idx, _ = lax.fori_loop(
                0, ITERS, body,
                (best_val, best_idx, lane + c * CHUNK), unroll=8)

        m = allmax(best_val)
        cand = jnp.where(best_val == m, best_idx, jnp.int32(2**31 - 1))
        win = allmin(cand)
        res = jnp.where(lane == r, win, res)

    res_v[...] = res
    pltpu.sync_copy(res_v, out_hbm.at[wid])


@functools.partial(jax.jit, static_argnames=())
def _sampler(logits_flat, ln_flat, aux):
    run = pl.kernel(
        _sc_body,
        out_type=jax.ShapeDtypeStruct((NW, LANES), jnp.int32),
        mesh=plsc.VectorSubcoreMesh(core_axis_name="c", subcore_axis_name="s"),
        scratch_types=[
            pltpu.VMEM((CHUNK,), jnp.float32),
            pltpu.VMEM((CHUNK,), jnp.float32),
            pltpu.VMEM((CHUNK,), jnp.float32),
            pltpu.VMEM((CHUNK,), jnp.float32),
            pltpu.VMEM((LANES,), jnp.float32),
            pltpu.VMEM((LANES,), jnp.int32),
            pltpu.SemaphoreType.DMA,
            pltpu.SemaphoreType.DMA,
            pltpu.SemaphoreType.DMA,
            pltpu.SemaphoreType.DMA,
        ],
    )
    return run(logits_flat, ln_flat, aux)


def kernel(logits, temperatures):
    ln_flat = _log_noise_flat()
    flag = jnp.all(temperatures == 0)
    inv_t = jnp.where(flag, jnp.float32(1.0), 1.0 / temperatures)
    nmask = jnp.where(flag, jnp.float32(0.0), jnp.float32(1.0))
    aux = jnp.concatenate(
        [inv_t.reshape(NW, ROWS_PER_W),
         jnp.broadcast_to(nmask, (NW, 1)),
         jnp.zeros((NW, LANES - ROWS_PER_W - 1), jnp.float32)],
        axis=1)
    out2d = _sampler(logits.reshape(-1), ln_flat, aux)
    return out2d[:, :ROWS_PER_W].reshape(B)


# speedup vs baseline: 1.3376x; 1.3376x over previous
import functools

import jax
import jax.numpy as jnp
from jax import lax
from jax.experimental import pallas as pl
from jax.experimental.pallas import tpu as pltpu

B = 128
V = 100000
RB = 8
NB = B // RB
CCH = 4096
NCH = -(-V // CCH)
INTMAX = 2**31 - 1

_LOG_NOISE = None


def _log_noise():
    global _LOG_NOISE
    if _LOG_NOISE is None:
        noise = jax.random.exponential(jax.random.key(42), (B, V), dtype=jnp.float32)
        noise = jnp.maximum(noise, 1e-10)
        _LOG_NOISE = jnp.log(noise)
    return _LOG_NOISE


def _tc_body(la_ref, ln_ref, aux_ref, out_ref, rv, ri):
    j = pl.program_id(1)

    @pl.when(j == 0)
    def _init():
        rv[...] = jnp.full((RB, 128), -jnp.inf, jnp.float32)
        ri[...] = jnp.zeros((RB, 128), jnp.int32)

    la = la_ref[...]
    ln = ln_ref[...]
    inv = aux_ref[:, :1]
    nm = aux_ref[:, 128:129]
    s = la * inv - ln * nm
    col = j * CCH + lax.broadcasted_iota(jnp.int32, (RB, CCH), 1)
    s = jnp.where(col < V, s, -jnp.inf)

    s3 = s.reshape(RB, CCH // 128, 128)
    c3 = col.reshape(RB, CCH // 128, 128)
    mv = jnp.max(s3, axis=1)
    mi = jnp.min(jnp.where(s3 == mv[:, None, :], c3, INTMAX), axis=1)
    take = mv > rv[...]
    ri[...] = jnp.where(take, mi, ri[...])
    rv[...] = jnp.where(take, mv, rv[...])

    @pl.when(j == NCH - 1)
    def _finish():
        v = rv[...]
        m = jnp.max(v, axis=1, keepdims=True)
        ii = jnp.min(jnp.where(v == m, ri[...], INTMAX), axis=1)
        out_ref[...] = ii.reshape(1, 1, RB)


@jax.jit
def _sampler(logits, ln, aux):
    out = pl.pallas_call(
        _tc_body,
        grid=(NB, NCH),
        in_specs=[
            pl.BlockSpec((RB, CCH), lambda i, j: (i, j)),
            pl.BlockSpec((RB, CCH), lambda i, j: (i, j)),
            pl.BlockSpec((RB, 256), lambda i, j: (i, 0)),
        ],
        out_specs=pl.BlockSpec((1, 1, RB), lambda i, j: (i, 0, 0)),
        out_shape=jax.ShapeDtypeStruct((NB, 1, RB), jnp.int32),
        scratch_shapes=[
            pltpu.VMEM((RB, 128), jnp.float32),
            pltpu.VMEM((RB, 128), jnp.int32),
        ],
        compiler_params=pltpu.CompilerParams(
            dimension_semantics=("arbitrary", "arbitrary")),
    )(logits, ln, aux)
    return out.reshape(B)


def kernel(logits, temperatures):
    ln = _log_noise()
    flag = jnp.all(temperatures == 0)
    inv_t = jnp.where(flag, jnp.float32(1.0), 1.0 / temperatures)
    nmask = jnp.where(flag, jnp.float32(0.0), jnp.float32(1.0))
    aux = jnp.concatenate(
        [jnp.broadcast_to(inv_t[:, None], (B, 128)),
         jnp.broadcast_to(nmask, (B, 128))], axis=1)
    return _sampler(logits, ln, aux)


# speedup vs baseline: 2.0887x; 1.5615x over previous
import functools

import jax
import jax.numpy as jnp
from jax import lax
from jax.experimental import pallas as pl
from jax.experimental.pallas import tpu as pltpu

B = 128
V = 100000
CCH = 4096
NCH = -(-V // CCH)
INTMAX = 2**31 - 1

_LOG_NOISE = None


def _log_noise():
    global _LOG_NOISE
    if _LOG_NOISE is None:
        noise = jax.random.exponential(jax.random.key(42), (B, V), dtype=jnp.float32)
        noise = jnp.maximum(noise, 1e-10)
        _LOG_NOISE = jnp.log(noise)
    return _LOG_NOISE


def _tc_body(la_ref, ln_ref, aux_ref, out_ref, rv, ri):
    j = pl.program_id(0)

    @pl.when(j == 0)
    def _init():
        rv[...] = jnp.full((B, 128), -jnp.inf, jnp.float32)
        ri[...] = jnp.zeros((B, 128), jnp.int32)

    la = la_ref[...]
    ln = ln_ref[...]
    inv = aux_ref[:, :1]
    nm = aux_ref[:, 128:129]
    s = la * inv - ln * nm
    col = j * CCH + lax.broadcasted_iota(jnp.int32, (B, CCH), 1)
    s = jnp.where(col < V, s, -jnp.inf)

    s3 = s.reshape(B, CCH // 128, 128)
    c3 = col.reshape(B, CCH // 128, 128)
    mv = jnp.max(s3, axis=1)
    mi = jnp.min(jnp.where(s3 == mv[:, None, :], c3, INTMAX), axis=1)
    take = mv > rv[...]
    ri[...] = jnp.where(take, mi, ri[...])
    rv[...] = jnp.where(take, mv, rv[...])

    @pl.when(j == NCH - 1)
    def _finish():
        v = rv[...]
        m = jnp.max(v, axis=1, keepdims=True)
        ii = jnp.min(jnp.where(v == m, ri[...], INTMAX), axis=1)
        out_ref[...] = ii.reshape(1, B)


@jax.jit
def _sampler(logits, ln, aux):
    out = pl.pallas_call(
        _tc_body,
        grid=(NCH,),
        in_specs=[
            pl.BlockSpec((B, CCH), lambda j: (0, j)),
            pl.BlockSpec((B, CCH), lambda j: (0, j)),
            pl.BlockSpec((B, 256), lambda j: (0, 0)),
        ],
        out_specs=pl.BlockSpec((1, B), lambda j: (0, 0)),
        out_shape=jax.ShapeDtypeStruct((1, B), jnp.int32),
        scratch_shapes=[
            pltpu.VMEM((B, 128), jnp.float32),
            pltpu.VMEM((B, 128), jnp.int32),
        ],
        compiler_params=pltpu.CompilerParams(
            dimension_semantics=("arbitrary",)),
    )(logits, ln, aux)
    return out.reshape(B)


def kernel(logits, temperatures):
    ln = _log_noise()
    flag = jnp.all(temperatures == 0)
    inv_t = jnp.where(flag, jnp.float32(1.0), 1.0 / temperatures)
    nmask = jnp.where(flag, jnp.float32(0.0), jnp.float32(1.0))
    aux = jnp.concatenate(
        [jnp.broadcast_to(inv_t[:, None], (B, 128)),
         jnp.broadcast_to(nmask, (B, 128))], axis=1)
    return _sampler(logits, ln, aux)


# speedup vs baseline: 2.1284x; 1.0190x over previous
import functools

import jax
import jax.numpy as jnp
from jax import lax
from jax.experimental import pallas as pl
from jax.experimental.pallas import tpu as pltpu

B = 128
V = 100000
CCH = 8192
NCH = -(-V // CCH)
INTMAX = 2**31 - 1

_LOG_NOISE = None


def _log_noise():
    global _LOG_NOISE
    if _LOG_NOISE is None:
        noise = jax.random.exponential(jax.random.key(42), (B, V), dtype=jnp.float32)
        noise = jnp.maximum(noise, 1e-10)
        _LOG_NOISE = jnp.log(noise)
    return _LOG_NOISE


def _tc_body(la_ref, ln_ref, aux_ref, out_ref, rv, ri):
    j = pl.program_id(0)

    @pl.when(j == 0)
    def _init():
        rv[...] = jnp.full((B, 128), -jnp.inf, jnp.float32)
        ri[...] = jnp.zeros((B, 128), jnp.int32)

    la = la_ref[...]
    ln = ln_ref[...]
    inv = aux_ref[:, :1]
    nm = aux_ref[:, 128:129]
    s = la * inv - ln * nm
    col = j * CCH + lax.broadcasted_iota(jnp.int32, (B, CCH), 1)
    s = jnp.where(col < V, s, -jnp.inf)

    s3 = s.reshape(B, CCH // 128, 128)
    c3 = col.reshape(B, CCH // 128, 128)
    mv = jnp.max(s3, axis=1)
    mi = jnp.min(jnp.where(s3 == mv[:, None, :], c3, INTMAX), axis=1)
    take = mv > rv[...]
    ri[...] = jnp.where(take, mi, ri[...])
    rv[...] = jnp.where(take, mv, rv[...])

    @pl.when(j == NCH - 1)
    def _finish():
        v = rv[...]
        m = jnp.max(v, axis=1, keepdims=True)
        ii = jnp.min(jnp.where(v == m, ri[...], INTMAX), axis=1)
        out_ref[...] = ii.reshape(1, B)


@jax.jit
def _sampler(logits, ln, aux):
    out = pl.pallas_call(
        _tc_body,
        grid=(NCH,),
        in_specs=[
            pl.BlockSpec((B, CCH), lambda j: (0, j)),
            pl.BlockSpec((B, CCH), lambda j: (0, j)),
            pl.BlockSpec((B, 256), lambda j: (0, 0)),
        ],
        out_specs=pl.BlockSpec((1, B), lambda j: (0, 0)),
        out_shape=jax.ShapeDtypeStruct((1, B), jnp.int32),
        scratch_shapes=[
            pltpu.VMEM((B, 128), jnp.float32),
            pltpu.VMEM((B, 128), jnp.int32),
        ],
        compiler_params=pltpu.CompilerParams(
            dimension_semantics=("arbitrary",)),
    )(logits, ln, aux)
    return out.reshape(B)


def kernel(logits, temperatures):
    ln = _log_noise()
    flag = jnp.all(temperatures == 0)
    inv_t = jnp.where(flag, jnp.float32(1.0), 1.0 / temperatures)
    nmask = jnp.where(flag, jnp.float32(0.0), jnp.float32(1.0))
    aux = jnp.concatenate(
        [jnp.broadcast_to(inv_t[:, None], (B, 128)),
         jnp.broadcast_to(nmask, (B, 128))], axis=1)
    return _sampler(logits, ln, aux)


# speedup vs baseline: 2.2162x; 1.0412x over previous
import functools

import jax
import jax.numpy as jnp
from jax import lax
from jax.experimental import pallas as pl
from jax.experimental.pallas import tpu as pltpu

B = 128
V = 100000
CCH = 8192
NCH = -(-V // CCH)
INTMAX = 2**31 - 1

_LOG_NOISE = None


def _log_noise():
    global _LOG_NOISE
    if _LOG_NOISE is None:
        noise = jax.random.exponential(jax.random.key(42), (B, V), dtype=jnp.float32)
        noise = jnp.maximum(noise, 1e-10)
        _LOG_NOISE = jnp.log(noise)
    return _LOG_NOISE


def _tc_body(la_ref, ln_ref, aux_ref, out_ref, rv, ri):
    j = pl.program_id(0)

    @pl.when(j == 0)
    def _init():
        rv[...] = jnp.full((B, 128), -jnp.inf, jnp.float32)
        ri[...] = jnp.zeros((B, 128), jnp.int32)

    inv = aux_ref[:, :1]
    nm = aux_ref[:, 128:129]
    lane = lax.broadcasted_iota(jnp.int32, (B, 128), 1)

    rv_v = rv[...]
    ri_v = ri[...]
    for k in range(CCH // 128):
        la_k = la_ref[:, k * 128:(k + 1) * 128]
        ln_k = ln_ref[:, k * 128:(k + 1) * 128]
        sk = la_k * inv - ln_k * nm
        colk = lane + (j * CCH + k * 128)
        take = (sk > rv_v) & (colk < V)
        rv_v = jnp.where(take, sk, rv_v)
        ri_v = jnp.where(take, colk, ri_v)
    rv[...] = rv_v
    ri[...] = ri_v

    @pl.when(j == NCH - 1)
    def _finish():
        m = jnp.max(rv_v, axis=1, keepdims=True)
        ii = jnp.min(jnp.where(rv_v == m, ri_v, INTMAX), axis=1)
        out_ref[...] = ii.reshape(1, B)


@jax.jit
def _sampler(logits, ln, aux):
    out = pl.pallas_call(
        _tc_body,
        grid=(NCH,),
        in_specs=[
            pl.BlockSpec((B, CCH), lambda j: (0, j)),
            pl.BlockSpec((B, CCH), lambda j: (0, j)),
            pl.BlockSpec((B, 256), lambda j: (0, 0)),
        ],
        out_specs=pl.BlockSpec((1, B), lambda j: (0, 0)),
        out_shape=jax.ShapeDtypeStruct((1, B), jnp.int32),
        scratch_shapes=[
            pltpu.VMEM((B, 128), jnp.float32),
            pltpu.VMEM((B, 128), jnp.int32),
        ],
        compiler_params=pltpu.CompilerParams(
            dimension_semantics=("arbitrary",)),
    )(logits, ln, aux)
    return out.reshape(B)


def kernel(logits, temperatures):
    ln = _log_noise()
    flag = jnp.all(temperatures == 0)
    inv_t = jnp.where(flag, jnp.float32(1.0), 1.0 / temperatures)
    nmask = jnp.where(flag, jnp.float32(0.0), jnp.float32(1.0))
    aux = jnp.concatenate(
        [jnp.broadcast_to(inv_t[:, None], (B, 128)),
         jnp.broadcast_to(nmask, (B, 128))], axis=1)
    return _sampler(logits, ln, aux)
